# Initial kernel scaffold; baseline (speedup 1.0000x reference)
import jax
import jax.numpy as jnp
from jax.experimental import pallas as pl


def kernel(v, e, edge_index, W1, b1, W2, b2):
    raise NotImplementedError("write your pallas kernel here")


# speedup vs baseline: 36.2828x; 36.2828x over previous
import functools

import jax
import jax.numpy as jnp
import numpy as np
from jax import lax
from jax.experimental import pallas as pl
from jax.experimental.pallas import tpu as pltpu
from jax.experimental.pallas import tpu_sc as plsc

HIDDEN = 256
NUM_FILTERS = 16
NUM_HEADS = 16
N_NODES = 10000
N_EDGES = 160000
SHIFT = float(np.log(2.0))

NC = 2
NS = 16
DH = HIDDEN // NC
BATCH = 80
EPS = N_EDGES // NS
NBLK = EPS // BATCH
STRIPE = 640
LAST_STRIPE = N_NODES - 15 * STRIPE


def _sc_segment_sum(e3, idx3, zrows):
    mesh = plsc.VectorSubcoreMesh(core_axis_name="c", subcore_axis_name="s")

    @functools.partial(
        pl.kernel,
        mesh=mesh,
        out_type=jax.ShapeDtypeStruct((NC, N_NODES, DH), jnp.float32),
        scratch_types=[
            pltpu.VMEM((NBLK, BATCH), jnp.int32),
            pltpu.VMEM((BATCH, DH), jnp.float32),
            pltpu.VMEM_SHARED((N_NODES, DH), jnp.float32),
        ],
    )
    def k(e_hbm, idx_hbm, z_hbm, out_hbm, idx_v, e_v, acc):
        c = lax.axis_index("c")
        s = lax.axis_index("s")
        @pl.when(s < NS - 1)
        def _():
            pltpu.sync_copy(z_hbm, acc.at[pl.ds(s * STRIPE, STRIPE)])

        @pl.when(s == NS - 1)
        def _():
            pltpu.sync_copy(z_hbm.at[pl.ds(0, LAST_STRIPE)],
                            acc.at[pl.ds(s * STRIPE, LAST_STRIPE)])

        pltpu.sync_copy(idx_hbm.at[s], idx_v)
        plsc.subcore_barrier()

        @pl.loop(0, NBLK)
        def _(j):
            base = s * EPS + j * BATCH
            pltpu.sync_copy(e_hbm.at[pl.ds(base, BATCH), c], e_v)
            pltpu.sync_copy(e_v, acc.at[idx_v.at[j]], add=True)

        plsc.subcore_barrier()

        @pl.when(s < NS - 1)
        def _():
            pltpu.sync_copy(acc.at[pl.ds(s * STRIPE, STRIPE)],
                            out_hbm.at[c, pl.ds(s * STRIPE, STRIPE)])

        @pl.when(s == NS - 1)
        def _():
            pltpu.sync_copy(acc.at[pl.ds(s * STRIPE, LAST_STRIPE)],
                            out_hbm.at[c, pl.ds(s * STRIPE, LAST_STRIPE)])

    return k(e3, idx3, zrows)


def _mlp_body(a0_ref, a1_ref, v_ref, w1a_ref, w1b_ref, b1_ref, w2_ref, b2_ref,
              o_ref):
    dn = (((1,), (1,)), ((), ()))
    h = (
        lax.dot_general(a0_ref[0], w1a_ref[...], dn,
                        preferred_element_type=jnp.float32,
                        precision=lax.Precision.HIGHEST)
        + lax.dot_general(a1_ref[0], w1b_ref[...], dn,
                          preferred_element_type=jnp.float32,
                          precision=lax.Precision.HIGHEST)
        + b1_ref[...]
    )
    sp = jnp.logaddexp(h, 0.0) - SHIFT
    o_ref[...] = (
        lax.dot_general(sp, w2_ref[...], dn,
                        preferred_element_type=jnp.float32,
                        precision=lax.Precision.HIGHEST)
        + b2_ref[...]
        + v_ref[...]
    )


def _mlp(agg, v, w1a, w1b, b1, w2, b2):
    rows = 1000
    grid = (N_NODES // rows,)
    return pl.pallas_call(
        _mlp_body,
        grid=grid,
        in_specs=[
            pl.BlockSpec((1, rows, DH), lambda i: (0, i, 0)),
            pl.BlockSpec((1, rows, DH), lambda i: (1, i, 0)),
            pl.BlockSpec((rows, HIDDEN), lambda i: (i, 0)),
            pl.BlockSpec((HIDDEN, DH), lambda i: (0, 0)),
            pl.BlockSpec((HIDDEN, DH), lambda i: (0, 0)),
            pl.BlockSpec((1, HIDDEN), lambda i: (0, 0)),
            pl.BlockSpec((HIDDEN, HIDDEN), lambda i: (0, 0)),
            pl.BlockSpec((1, HIDDEN), lambda i: (0, 0)),
        ],
        out_specs=pl.BlockSpec((rows, HIDDEN), lambda i: (i, 0)),
        out_shape=jax.ShapeDtypeStruct((N_NODES, HIDDEN), jnp.float32),
    )(agg, agg, v, w1a, w1b, b1, w2, b2)


def kernel(v, e, edge_index, W1, b1, W2, b2):
    idx3 = edge_index[1].astype(jnp.int32).reshape(NS, NBLK, BATCH)
    e3 = e.reshape(N_EDGES, NC, DH)
    zrows = jnp.zeros((STRIPE, DH), jnp.float32)
    agg = _sc_segment_sum(e3, idx3, zrows)
    return _mlp(agg, v, W1[:, :DH], W1[:, DH:], b1.reshape(1, HIDDEN), W2,
                b2.reshape(1, HIDDEN))


# speedup vs baseline: 43.4167x; 1.1966x over previous
import functools

import jax
import jax.numpy as jnp
import numpy as np
from jax import lax
from jax.experimental import pallas as pl
from jax.experimental.pallas import tpu as pltpu
from jax.experimental.pallas import tpu_sc as plsc

HIDDEN = 256
NUM_FILTERS = 16
NUM_HEADS = 16
N_NODES = 10000
N_EDGES = 160000
SHIFT = float(np.log(2.0))

NC = 2
NS = 16
DH = HIDDEN // NC
BATCH = 80
EPS = N_EDGES // NS
NBLK = EPS // BATCH
GROW = 80
SPG = GROW // BATCH
NSB = EPS // GROW
STRIPE = 640
LAST_STRIPE = N_NODES - 15 * STRIPE


def _sc_segment_sum(e3, idx3, zrows):
    mesh = plsc.VectorSubcoreMesh(core_axis_name="c", subcore_axis_name="s")

    @functools.partial(
        pl.kernel,
        mesh=mesh,
        out_type=jax.ShapeDtypeStruct((NC, N_NODES, DH), jnp.float32),
        scratch_types=[
            pltpu.VMEM((NBLK, BATCH), jnp.int32),
            pltpu.VMEM((GROW, DH), jnp.float32),
            pltpu.VMEM((GROW, DH), jnp.float32),
            pltpu.VMEM_SHARED((N_NODES, DH), jnp.float32),
            pltpu.SemaphoreType.DMA,
            pltpu.SemaphoreType.DMA,
        ],
    )
    def k(e_hbm, idx_hbm, z_hbm, out_hbm, idx_v, e_v0, e_v1, acc, sem0, sem1):
        c = lax.axis_index("c")
        s = lax.axis_index("s")
        @pl.when(s < NS - 1)
        def _():
            pltpu.sync_copy(z_hbm, acc.at[pl.ds(s * STRIPE, STRIPE)])

        @pl.when(s == NS - 1)
        def _():
            pltpu.sync_copy(z_hbm.at[pl.ds(0, LAST_STRIPE)],
                            acc.at[pl.ds(s * STRIPE, LAST_STRIPE)])

        pltpu.sync_copy(idx_hbm.at[s], idx_v)
        plsc.subcore_barrier()

        def gather(j, buf, sem):
            base = s * EPS + j * GROW
            return pltpu.make_async_copy(
                e_hbm.at[pl.ds(base, GROW), c], buf, sem)

        def scatter(j, buf):
            for t in range(SPG):
                pltpu.sync_copy(buf.at[pl.ds(t * BATCH, BATCH)],
                                acc.at[idx_v.at[j * SPG + t]], add=True)

        gather(0, e_v0, sem0).start()

        @pl.loop(0, NSB - 1, step=2)
        def _(j):
            gather(j + 1, e_v1, sem1).start()
            gather(j, e_v0, sem0).wait()
            scatter(j, e_v0)
            gather(j + 2, e_v0, sem0).start()
            gather(j + 1, e_v1, sem1).wait()
            scatter(j + 1, e_v1)

        gather(NSB - 1, e_v0, sem0).wait()
        scatter(NSB - 1, e_v0)

        plsc.subcore_barrier()

        @pl.when(s < NS - 1)
        def _():
            pltpu.sync_copy(acc.at[pl.ds(s * STRIPE, STRIPE)],
                            out_hbm.at[c, pl.ds(s * STRIPE, STRIPE)])

        @pl.when(s == NS - 1)
        def _():
            pltpu.sync_copy(acc.at[pl.ds(s * STRIPE, LAST_STRIPE)],
                            out_hbm.at[c, pl.ds(s * STRIPE, LAST_STRIPE)])

    return k(e3, idx3, zrows)


def _mlp_body(a0_ref, a1_ref, v_ref, w1a_ref, w1b_ref, b1_ref, w2_ref, b2_ref,
              o_ref):
    dn = (((1,), (1,)), ((), ()))
    h = (
        lax.dot_general(a0_ref[0], w1a_ref[...], dn,
                        preferred_element_type=jnp.float32,
                        precision=lax.Precision.HIGHEST)
        + lax.dot_general(a1_ref[0], w1b_ref[...], dn,
                          preferred_element_type=jnp.float32,
                          precision=lax.Precision.HIGHEST)
        + b1_ref[...]
    )
    sp = jnp.logaddexp(h, 0.0) - SHIFT
    o_ref[...] = (
        lax.dot_general(sp, w2_ref[...], dn,
                        preferred_element_type=jnp.float32,
                        precision=lax.Precision.HIGHEST)
        + b2_ref[...]
        + v_ref[...]
    )


def _mlp(agg, v, w1a, w1b, b1, w2, b2):
    rows = 1000
    grid = (N_NODES // rows,)
    return pl.pallas_call(
        _mlp_body,
        grid=grid,
        in_specs=[
            pl.BlockSpec((1, rows, DH), lambda i: (0, i, 0)),
            pl.BlockSpec((1, rows, DH), lambda i: (1, i, 0)),
            pl.BlockSpec((rows, HIDDEN), lambda i: (i, 0)),
            pl.BlockSpec((HIDDEN, DH), lambda i: (0, 0)),
            pl.BlockSpec((HIDDEN, DH), lambda i: (0, 0)),
            pl.BlockSpec((1, HIDDEN), lambda i: (0, 0)),
            pl.BlockSpec((HIDDEN, HIDDEN), lambda i: (0, 0)),
            pl.BlockSpec((1, HIDDEN), lambda i: (0, 0)),
        ],
        out_specs=pl.BlockSpec((rows, HIDDEN), lambda i: (i, 0)),
        out_shape=jax.ShapeDtypeStruct((N_NODES, HIDDEN), jnp.float32),
    )(agg, agg, v, w1a, w1b, b1, w2, b2)


def kernel(v, e, edge_index, W1, b1, W2, b2):
    idx3 = edge_index[1].astype(jnp.int32).reshape(NS, NBLK, BATCH)
    e3 = e.reshape(N_EDGES, NC, DH)
    zrows = jnp.zeros((STRIPE, DH), jnp.float32)
    agg = _sc_segment_sum(e3, idx3, zrows)
    return _mlp(agg, v, W1[:, :DH], W1[:, DH:], b1.reshape(1, HIDDEN), W2,
                b2.reshape(1, HIDDEN))


# speedup vs baseline: 57.6731x; 1.3284x over previous
import functools

import jax
import jax.numpy as jnp
import numpy as np
from jax import lax
from jax.experimental import pallas as pl
from jax.experimental.pallas import tpu as pltpu
from jax.experimental.pallas import tpu_sc as plsc

HIDDEN = 256
NUM_FILTERS = 16
NUM_HEADS = 16
N_NODES = 10000
N_EDGES = 160000
SHIFT = float(np.log(2.0))

NC = 2
NS = 16
DH = HIDDEN // NC
BATCH = 80
EPS = N_EDGES // NS
NBLK = EPS // BATCH
GROW = 80
SPG = GROW // BATCH
NSB = EPS // GROW
STRIPE = 640
LAST_STRIPE = N_NODES - 15 * STRIPE


def _sc_segment_sum(e2, idx3, zrows):
    mesh = plsc.VectorSubcoreMesh(core_axis_name="c", subcore_axis_name="s")

    @functools.partial(
        pl.kernel,
        mesh=mesh,
        out_type=jax.ShapeDtypeStruct((NC, N_NODES, DH), jnp.float32),
        scratch_types=[
            pltpu.VMEM((NBLK, BATCH), jnp.int32),
            pltpu.VMEM((GROW, DH), jnp.float32),
            pltpu.VMEM((GROW, DH), jnp.float32),
            pltpu.VMEM_SHARED((N_NODES, DH), jnp.float32),
            pltpu.SemaphoreType.DMA,
            pltpu.SemaphoreType.DMA,
        ],
    )
    def k(e_hbm, idx_hbm, z_hbm, out_hbm, idx_v, e_v0, e_v1, acc, sem0, sem1):
        c = lax.axis_index("c")
        s = lax.axis_index("s")
        @pl.when(s < NS - 1)
        def _():
            pltpu.sync_copy(z_hbm, acc.at[pl.ds(s * STRIPE, STRIPE)])

        @pl.when(s == NS - 1)
        def _():
            pltpu.sync_copy(z_hbm.at[pl.ds(0, LAST_STRIPE)],
                            acc.at[pl.ds(s * STRIPE, LAST_STRIPE)])

        pltpu.sync_copy(idx_hbm.at[s], idx_v)
        plsc.subcore_barrier()

        def gather(j, buf, sem):
            base = s * EPS + j * GROW
            return pltpu.make_async_copy(
                e_hbm.at[pl.ds(base, GROW), pl.ds(c * DH, DH)], buf, sem)

        def scatter(j, buf):
            for t in range(SPG):
                pltpu.sync_copy(buf.at[pl.ds(t * BATCH, BATCH)],
                                acc.at[idx_v.at[j * SPG + t]], add=True)

        gather(0, e_v0, sem0).start()

        @pl.loop(0, NSB - 1, step=2)
        def _(j):
            gather(j + 1, e_v1, sem1).start()
            gather(j, e_v0, sem0).wait()
            scatter(j, e_v0)
            gather(j + 2, e_v0, sem0).start()
            gather(j + 1, e_v1, sem1).wait()
            scatter(j + 1, e_v1)

        gather(NSB - 1, e_v0, sem0).wait()
        scatter(NSB - 1, e_v0)

        plsc.subcore_barrier()

        @pl.when(s < NS - 1)
        def _():
            pltpu.sync_copy(acc.at[pl.ds(s * STRIPE, STRIPE)],
                            out_hbm.at[c, pl.ds(s * STRIPE, STRIPE)])

        @pl.when(s == NS - 1)
        def _():
            pltpu.sync_copy(acc.at[pl.ds(s * STRIPE, LAST_STRIPE)],
                            out_hbm.at[c, pl.ds(s * STRIPE, LAST_STRIPE)])

    return k(e2, idx3, zrows)


def _mlp_body(a0_ref, a1_ref, v_ref, w1a_ref, w1b_ref, b1_ref, w2_ref, b2_ref,
              o_ref):
    dn = (((1,), (1,)), ((), ()))
    h = (
        lax.dot_general(a0_ref[0], w1a_ref[...], dn,
                        preferred_element_type=jnp.float32,
                        precision=lax.Precision.HIGHEST)
        + lax.dot_general(a1_ref[0], w1b_ref[...], dn,
                          preferred_element_type=jnp.float32,
                          precision=lax.Precision.HIGHEST)
        + b1_ref[...]
    )
    sp = jnp.logaddexp(h, 0.0) - SHIFT
    o_ref[...] = (
        lax.dot_general(sp, w2_ref[...], dn,
                        preferred_element_type=jnp.float32,
                        precision=lax.Precision.HIGHEST)
        + b2_ref[...]
        + v_ref[...]
    )


def _mlp(agg, v, w1a, w1b, b1, w2, b2):
    rows = 1000
    grid = (N_NODES // rows,)
    return pl.pallas_call(
        _mlp_body,
        grid=grid,
        in_specs=[
            pl.BlockSpec((1, rows, DH), lambda i: (0, i, 0)),
            pl.BlockSpec((1, rows, DH), lambda i: (1, i, 0)),
            pl.BlockSpec((rows, HIDDEN), lambda i: (i, 0)),
            pl.BlockSpec((HIDDEN, DH), lambda i: (0, 0)),
            pl.BlockSpec((HIDDEN, DH), lambda i: (0, 0)),
            pl.BlockSpec((1, HIDDEN), lambda i: (0, 0)),
            pl.BlockSpec((HIDDEN, HIDDEN), lambda i: (0, 0)),
            pl.BlockSpec((1, HIDDEN), lambda i: (0, 0)),
        ],
        out_specs=pl.BlockSpec((rows, HIDDEN), lambda i: (i, 0)),
        out_shape=jax.ShapeDtypeStruct((N_NODES, HIDDEN), jnp.float32),
    )(agg, agg, v, w1a, w1b, b1, w2, b2)


def kernel(v, e, edge_index, W1, b1, W2, b2):
    idx3 = edge_index[1].astype(jnp.int32).reshape(NS, NBLK, BATCH)
    e2 = e.reshape(N_EDGES, HIDDEN)
    zrows = jnp.zeros((STRIPE, DH), jnp.float32)
    agg = _sc_segment_sum(e2, idx3, zrows)
    return _mlp(agg, v, W1[:, :DH], W1[:, DH:], b1.reshape(1, HIDDEN), W2,
                b2.reshape(1, HIDDEN))
